# baseline (speedup 1.0000x reference)
import functools

import jax
import jax.numpy as jnp
from jax import lax
from jax.experimental import pallas as pl
from jax.experimental.pallas import tpu as pltpu
from jax.experimental.pallas import tpu_sc as plsc

N = 10000
E = 320000
D = 128

NC = 2
NS = 16
NW = NC * NS

CH = 128
EROWS = E // CH
SLAB = 8
FULL_G = EROWS // SLAB

NPAD = 10240
DEG_PER_TILE = NPAD // NS
ROWS_PER_TILE = NPAD // NS


def _n_groups(wid):
  return jnp.where(wid < 24, 10, 9)


def _group0(wid):
  return jnp.where(wid < 24, 10 * wid, 240 + 9 * (wid - 24))


def _zero_1d(ref, nwords):
  zv = jnp.zeros((16,), jnp.float32)

  def body(i, _):
    ref[pl.ds(i * 16, 16)] = zv
    return 0

  lax.fori_loop(0, nwords // 16, body, 0)


def _zero_2d(ref, nrows):
  zv = jnp.zeros((16,), jnp.float32)

  def body(i, _):
    def inner(j, _):
      ref[i, pl.ds(j * 16, 16)] = zv
      return 0

    lax.fori_loop(0, 8, inner, 0)
    return 0

  lax.fori_loop(0, nrows, body, 0)


def _load_tile_rows(src_ref, dst_ref, sidx, didx, sx, dx, wid, row0):
  pltpu.sync_copy(src_ref.at[pl.ds(row0, 72)], sidx.at[pl.ds(0, 72)])
  pltpu.sync_copy(dst_ref.at[pl.ds(row0, 72)], didx.at[pl.ds(0, 72)])

  @pl.when(wid < 24)
  def _load_rest():
    r = pl.multiple_of(row0 + 72, 8)
    pltpu.sync_copy(src_ref.at[pl.ds(r, 8)], sidx.at[pl.ds(72, 8)])
    pltpu.sync_copy(dst_ref.at[pl.ds(r, 8)], didx.at[pl.ds(72, 8)])

  @pl.when(wid < 4)
  def _load_extra():
    pltpu.sync_copy(src_ref.at[pl.ds(FULL_G * SLAB, 4)], sx)
    pltpu.sync_copy(dst_ref.at[pl.ds(FULL_G * SLAB, 4)], dx)


def _deg_body(src_ref, dst_ref, out_ref, sidx, didx, sx, dx, ones_v, zb,
              sems, dsrc_sh, ddst_sh):
  cid = lax.axis_index("c")
  sid = lax.axis_index("s")
  wid = sid * NC + cid

  ov = jnp.ones((16,), jnp.float32)
  for k in range(CH // 16):
    ones_v[pl.ds(k * 16, 16)] = ov

  n_g = _n_groups(wid)
  row0 = pl.multiple_of(_group0(wid) * SLAB, 8)
  _load_tile_rows(src_ref, dst_ref, sidx, didx, sx, dx, wid, row0)

  _zero_1d(zb, 640)
  off = pl.multiple_of(sid * DEG_PER_TILE, 8)
  pltpu.sync_copy(zb.at[pl.ds(0, DEG_PER_TILE)],
                  dsrc_sh.at[pl.ds(off, DEG_PER_TILE)])
  pltpu.sync_copy(zb.at[pl.ds(0, DEG_PER_TILE)],
                  ddst_sh.at[pl.ds(off, DEG_PER_TILE)])
  plsc.subcore_barrier()

  def group(g, _):
    hs = []
    for j in range(SLAB):
      c = g * SLAB + j
      hs.append(pltpu.async_copy(ones_v, dsrc_sh.at[sidx.at[c]],
                                 sems.at[j], add=True))
      hs.append(pltpu.async_copy(ones_v, ddst_sh.at[didx.at[c]],
                                 sems.at[SLAB + j], add=True))
    for h in hs:
      h.wait()
    return 0

  lax.fori_loop(0, n_g, group, 0)

  @pl.when(wid < 4)
  def _extra_chunk():
    pltpu.sync_copy(ones_v, dsrc_sh.at[sx.at[wid]], add=True)
    pltpu.sync_copy(ones_v, ddst_sh.at[dx.at[wid]], add=True)

  plsc.subcore_barrier()

  pltpu.sync_copy(dsrc_sh.at[pl.ds(off, DEG_PER_TILE)],
                  out_ref.at[cid, 0, pl.ds(off, DEG_PER_TILE)])
  pltpu.sync_copy(ddst_sh.at[pl.ds(off, DEG_PER_TILE)],
                  out_ref.at[cid, 1, pl.ds(off, DEG_PER_TILE)])


_deg_kernel = pl.kernel(
    _deg_body,
    out_type=jax.ShapeDtypeStruct((NC, 2, NPAD), jnp.float32),
    mesh=plsc.VectorSubcoreMesh(core_axis_name="c", subcore_axis_name="s"),
    scratch_types=[
        pltpu.VMEM((80, CH), jnp.int32),
        pltpu.VMEM((80, CH), jnp.int32),
        pltpu.VMEM((4, CH), jnp.int32),
        pltpu.VMEM((4, CH), jnp.int32),
        pltpu.VMEM((CH,), jnp.float32),
        pltpu.VMEM((640,), jnp.float32),
        pltpu.SemaphoreType.DMA((2 * SLAB,)),
        pltpu.VMEM_SHARED((NPAD,), jnp.float32),
        pltpu.VMEM_SHARED((NPAD,), jnp.float32),
    ],
)


_ACC_SLABS = ((0, 128), (128, 128), (256, 128), (384, 128), (512, 128))


def _scatter_body(p_ref, src_ref, dst_ref, out_ref, sidx, didx, sx, dx,
                  rows_v, isem, gsem, ssem, acc_sh):
  cid = lax.axis_index("c")
  sid = lax.axis_index("s")
  wid = sid * NC + cid

  n_g = _n_groups(wid)
  row0 = pl.multiple_of(_group0(wid) * SLAB, 8)

  pltpu.sync_copy(dst_ref.at[pl.ds(row0, 72)], didx.at[pl.ds(0, 72)])

  @pl.when(wid < 24)
  def _load_rest():
    r = pl.multiple_of(row0 + 72, 8)
    pltpu.sync_copy(dst_ref.at[pl.ds(r, 8)], didx.at[pl.ds(72, 8)])

  @pl.when(wid < 4)
  def _load_extra():
    pltpu.sync_copy(src_ref.at[pl.ds(FULL_G * SLAB, 4)], sx)
    pltpu.sync_copy(dst_ref.at[pl.ds(FULL_G * SLAB, 4)], dx)

  pltpu.sync_copy(src_ref.at[pl.ds(row0, SLAB)], sidx.at[0])

  _zero_2d(rows_v.at[0], 128)
  for r0, nr in _ACC_SLABS:
    dst_row = pl.multiple_of(sid * ROWS_PER_TILE + r0, 8)
    pltpu.sync_copy(rows_v.at[0].at[pl.ds(0, nr)],
                    acc_sh.at[pl.ds(dst_row, nr)])
  plsc.subcore_barrier()

  def _wait_scatter(c, par):
    pltpu.make_async_copy(rows_v.at[par], acc_sh.at[didx.at[c]],
                          ssem.at[par]).wait()

  def _wait_gather(s, j, par):
    pltpu.make_async_copy(p_ref.at[sidx.at[s % 2, j]], rows_v.at[par],
                          gsem.at[par]).wait()

  def slab_body(s, _):
    ps = s % 2

    @pl.when(s > 0)
    def _wait_idx():
      pltpu.make_async_copy(src_ref.at[pl.ds(row0 + s * SLAB, SLAB)],
                            sidx.at[ps], isem.at[ps]).wait()

    for j in range(SLAB):
      b = j % 2
      c = s * SLAB + j

      @pl.when(c >= 2)
      def _ws(c=c, b=b):
        _wait_scatter(c - 2, b)

      pltpu.async_copy(p_ref.at[sidx.at[ps, j]], rows_v.at[b], gsem.at[b])

      @pl.when(c >= 1)
      def _wg(s=s, j=j, c=c, b=b):
        if j == 0:
          _wait_gather(s - 1, SLAB - 1, 1 - b)
        else:
          _wait_gather(s, j - 1, 1 - b)
        pltpu.async_copy(rows_v.at[1 - b], acc_sh.at[didx.at[c - 1]],
                         ssem.at[1 - b], add=True)

      if j == 0:
        @pl.when(s < n_g - 1)
        def _prefetch_idx():
          pltpu.async_copy(src_ref.at[pl.ds(row0 + (s + 1) * SLAB, SLAB)],
                           sidx.at[1 - ps], isem.at[1 - ps])

    return 0

  lax.fori_loop(0, n_g, slab_body, 0)

  last = n_g * SLAB - 1
  _wait_gather(n_g - 1, SLAB - 1, 1)
  pltpu.sync_copy(rows_v.at[1], acc_sh.at[didx.at[last]], add=True)
  _wait_scatter(last - 1, 0)

  @pl.when(wid < 4)
  def _extra_chunk():
    pltpu.sync_copy(p_ref.at[sx.at[wid]], rows_v.at[0])
    pltpu.sync_copy(rows_v.at[0], acc_sh.at[dx.at[wid]], add=True)

  plsc.subcore_barrier()

  for r0, nr in _ACC_SLABS:
    row = pl.multiple_of(sid * ROWS_PER_TILE + r0, 8)
    pltpu.sync_copy(acc_sh.at[pl.ds(row, nr)],
                    out_ref.at[cid, pl.ds(row, nr)])


_scatter_kernel = pl.kernel(
    _scatter_body,
    out_type=jax.ShapeDtypeStruct((NC, NPAD, D), jnp.float32),
    mesh=plsc.VectorSubcoreMesh(core_axis_name="c", subcore_axis_name="s"),
    scratch_types=[
        pltpu.VMEM((2, SLAB, CH), jnp.int32),
        pltpu.VMEM((80, CH), jnp.int32),
        pltpu.VMEM((4, CH), jnp.int32),
        pltpu.VMEM((4, CH), jnp.int32),
        pltpu.VMEM((2, CH, D), jnp.float32),
        pltpu.SemaphoreType.DMA((2,)),
        pltpu.SemaphoreType.DMA((2,)),
        pltpu.SemaphoreType.DMA((2,)),
        pltpu.VMEM_SHARED((NPAD, D), jnp.float32),
    ],
)


def _dense_body(x_ref, w_ref, gamma_ref, beta_ref, deg_ref, p_ref):
  x = x_ref[...]
  mean = jnp.mean(x, axis=0)
  var = jnp.mean((x - mean) ** 2, axis=0)
  h = (x - mean) * lax.rsqrt(var + 1e-5) * gamma_ref[...] + beta_ref[...]
  h = jnp.maximum(h, 0.0)
  deg_src = deg_ref[0, 0, :] + deg_ref[1, 0, :]
  norm_src = jnp.where(deg_src > 0.0, lax.rsqrt(jnp.maximum(deg_src, 1.0)), 0.0)
  h = h * norm_src[:N, None]
  p_ref[...] = jnp.dot(h, w_ref[...], preferred_element_type=jnp.float32)


def _dense_kernel(x, W, gamma, beta, deg):
  return pl.pallas_call(
      _dense_body,
      out_shape=jax.ShapeDtypeStruct((N, D), jnp.float32),
  )(x, W, gamma, beta, deg)


def _final_body(x_ref, acc_ref, deg_ref, b_ref, out_ref):
  deg_dst = deg_ref[0, 1, :] + deg_ref[1, 1, :]
  norm_dst = jnp.where(deg_dst > 0.0, lax.rsqrt(jnp.maximum(deg_dst, 1.0)), 0.0)
  agg = acc_ref[0, :N] + acc_ref[1, :N]
  out_ref[...] = x_ref[...] + agg * norm_dst[:N, None] + b_ref[...]


def _final_kernel(x, acc, deg, b):
  return pl.pallas_call(
      _final_body,
      out_shape=jax.ShapeDtypeStruct((N, D), jnp.float32),
  )(x, acc, deg, b)


@jax.jit
def kernel(node_feats, edge_index, W, b, gamma, beta):
  ei = edge_index.astype(jnp.int32)
  src2 = ei[0].reshape(EROWS, CH)
  dst2 = ei[1].reshape(EROWS, CH)
  deg = _deg_kernel(src2, dst2)
  p = _dense_kernel(node_feats, W, gamma, beta, deg)
  acc = _scatter_kernel(p, src2, dst2)
  return _final_kernel(node_feats, acc, deg, b)


# speedup vs baseline: 1.0099x; 1.0099x over previous
import functools

import jax
import jax.numpy as jnp
from jax import lax
from jax.experimental import pallas as pl
from jax.experimental.pallas import tpu as pltpu
from jax.experimental.pallas import tpu_sc as plsc

N = 10000
E = 320000
D = 128

NC = 2
NS = 16
NW = NC * NS

NPAD = 10240
CH = 128
NCHUNK = 80
E_PER_W = NCHUNK * CH
EPAD = NW * E_PER_W

NSLAB = 10
SLAB = NCHUNK // NSLAB

DEG_PER_TILE = NPAD // NS
ROWS_PER_TILE = NPAD // NS
ZROWS = 128


def _zero_1d(ref, nwords):
  zv = jnp.zeros((16,), jnp.float32)

  def body(i, _):
    ref[pl.ds(i * 16, 16)] = zv
    return 0

  lax.fori_loop(0, nwords // 16, body, 0)


def _zero_2d(ref, nrows):
  zv = jnp.zeros((16,), jnp.float32)

  def body(i, _):
    def inner(j, _):
      ref[i, pl.ds(j * 16, 16)] = zv
      return 0

    lax.fori_loop(0, 8, inner, 0)
    return 0

  lax.fori_loop(0, nrows, body, 0)


EROWS = E // CH
FULL_G = 312


def _deg_body(src_ref, dst_ref, out_ref, sidx, didx, sx, dx, ones_v, zb,
              sems, dsrc_sh, ddst_sh):
  cid = lax.axis_index("c")
  sid = lax.axis_index("s")
  wid = sid * NC + cid

  ov = jnp.ones((16,), jnp.float32)
  for k in range(CH // 16):
    ones_v[pl.ds(k * 16, 16)] = ov

  lt24 = wid < 24
  n_g = jnp.where(lt24, 10, 9)
  g0 = jnp.where(lt24, 10 * wid, 240 + 9 * (wid - 24))
  row0 = pl.multiple_of(g0 * 8, 8)

  pltpu.sync_copy(src_ref.at[pl.ds(row0, 72)], sidx.at[pl.ds(0, 72)])
  pltpu.sync_copy(dst_ref.at[pl.ds(row0, 72)], didx.at[pl.ds(0, 72)])

  @pl.when(lt24)
  def _load_rest():
    r = pl.multiple_of(row0 + 72, 8)
    pltpu.sync_copy(src_ref.at[pl.ds(r, 8)], sidx.at[pl.ds(72, 8)])
    pltpu.sync_copy(dst_ref.at[pl.ds(r, 8)], didx.at[pl.ds(72, 8)])

  @pl.when(wid < 4)
  def _load_extra():
    pltpu.sync_copy(src_ref.at[pl.ds(FULL_G * 8, 4)], sx)
    pltpu.sync_copy(dst_ref.at[pl.ds(FULL_G * 8, 4)], dx)

  _zero_1d(zb, DEG_PER_TILE)
  off = pl.multiple_of(sid * DEG_PER_TILE, 8)
  pltpu.sync_copy(zb, dsrc_sh.at[pl.ds(off, DEG_PER_TILE)])
  pltpu.sync_copy(zb, ddst_sh.at[pl.ds(off, DEG_PER_TILE)])
  plsc.subcore_barrier()

  def group(g, _):
    hs = []
    for j in range(SLAB):
      c = g * SLAB + j
      hs.append(pltpu.async_copy(ones_v, dsrc_sh.at[sidx.at[c]],
                                 sems.at[j], add=True))
      hs.append(pltpu.async_copy(ones_v, ddst_sh.at[didx.at[c]],
                                 sems.at[SLAB + j], add=True))
    for h in hs:
      h.wait()
    return 0

  lax.fori_loop(0, n_g, group, 0)

  @pl.when(wid < 4)
  def _extra_chunk():
    pltpu.sync_copy(ones_v, dsrc_sh.at[sx.at[wid]], add=True)
    pltpu.sync_copy(ones_v, ddst_sh.at[dx.at[wid]], add=True)

  plsc.subcore_barrier()

  pltpu.sync_copy(dsrc_sh.at[pl.ds(off, DEG_PER_TILE)],
                  out_ref.at[cid, 0, pl.ds(off, DEG_PER_TILE)])
  pltpu.sync_copy(ddst_sh.at[pl.ds(off, DEG_PER_TILE)],
                  out_ref.at[cid, 1, pl.ds(off, DEG_PER_TILE)])


_deg_kernel = pl.kernel(
    _deg_body,
    out_type=jax.ShapeDtypeStruct((NC, 2, NPAD), jnp.float32),
    mesh=plsc.VectorSubcoreMesh(core_axis_name="c", subcore_axis_name="s"),
    scratch_types=[
        pltpu.VMEM((NCHUNK, CH), jnp.int32),
        pltpu.VMEM((NCHUNK, CH), jnp.int32),
        pltpu.VMEM((4, CH), jnp.int32),
        pltpu.VMEM((4, CH), jnp.int32),
        pltpu.VMEM((CH,), jnp.float32),
        pltpu.VMEM((DEG_PER_TILE,), jnp.float32),
        pltpu.SemaphoreType.DMA((2 * SLAB,)),
        pltpu.VMEM_SHARED((NPAD,), jnp.float32),
        pltpu.VMEM_SHARED((NPAD,), jnp.float32),
    ],
)


def _scatter_body(p_ref, src_ref, dst_ref, out_ref, sidx, didx, rows_v,
                  isem, gsem, ssem, acc_sh):
  cid = lax.axis_index("c")
  sid = lax.axis_index("s")
  wid = sid * NC + cid

  pltpu.sync_copy(dst_ref.at[wid], didx)
  pltpu.sync_copy(src_ref.at[wid, pl.ds(0, SLAB)], sidx.at[0])

  _zero_2d(rows_v.at[0], ZROWS)
  for k in range(ROWS_PER_TILE // ZROWS):
    r0 = pl.multiple_of(sid * ROWS_PER_TILE + k * ZROWS, 8)
    pltpu.sync_copy(rows_v.at[0], acc_sh.at[pl.ds(r0, ZROWS)])
  plsc.subcore_barrier()

  def _wait_scatter(c):
    pltpu.make_async_copy(rows_v.at[c % 2],
                          acc_sh.at[didx.at[c]],
                          ssem.at[c % 2]).wait()

  def _wait_gather(s, j):
    c = s * SLAB + j
    pltpu.make_async_copy(p_ref.at[sidx.at[s % 2, j]],
                          rows_v.at[c % 2],
                          gsem.at[c % 2]).wait()

  def slab_body(s, _):
    ps = s % 2

    @pl.when(s > 0)
    def _wait_idx():
      pltpu.make_async_copy(src_ref.at[wid, pl.ds(s * SLAB, SLAB)],
                            sidx.at[ps], isem.at[ps]).wait()

    for j in range(SLAB):
      b = j % 2
      c = s * SLAB + j

      @pl.when(c >= 2)
      def _ws(c=c):
        _wait_scatter(c - 2)

      pltpu.async_copy(p_ref.at[sidx.at[ps, j]], rows_v.at[b], gsem.at[b])

      @pl.when(c >= 1)
      def _wg(s=s, j=j, c=c):
        if j == 0:
          _wait_gather(s - 1, SLAB - 1)
        else:
          _wait_gather(s, j - 1)
        pltpu.async_copy(rows_v.at[1 - b], acc_sh.at[didx.at[c - 1]],
                         ssem.at[(c - 1) % 2], add=True)

      if j == 0:
        @pl.when(s < NSLAB - 1)
        def _prefetch_idx():
          pltpu.async_copy(src_ref.at[wid, pl.ds((s + 1) * SLAB, SLAB)],
                           sidx.at[1 - ps], isem.at[1 - ps])

    return 0

  lax.fori_loop(0, NSLAB, slab_body, 0)

  last = NCHUNK - 1
  _wait_gather(NSLAB - 1, SLAB - 1)
  pltpu.sync_copy(rows_v.at[last % 2], acc_sh.at[didx.at[last]], add=True)
  _wait_scatter(last - 1)
  plsc.subcore_barrier()

  for k in range(ROWS_PER_TILE // ZROWS):
    r0 = pl.multiple_of(sid * ROWS_PER_TILE + k * ZROWS, 8)
    pltpu.sync_copy(acc_sh.at[pl.ds(r0, ZROWS)],
                    out_ref.at[cid, pl.ds(r0, ZROWS)])


_scatter_kernel = pl.kernel(
    _scatter_body,
    out_type=jax.ShapeDtypeStruct((NC, NPAD, D), jnp.float32),
    mesh=plsc.VectorSubcoreMesh(core_axis_name="c", subcore_axis_name="s"),
    scratch_types=[
        pltpu.VMEM((2, SLAB, CH), jnp.int32),
        pltpu.VMEM((NCHUNK, CH), jnp.int32),
        pltpu.VMEM((2, ZROWS, D), jnp.float32),
        pltpu.SemaphoreType.DMA((2,)),
        pltpu.SemaphoreType.DMA((2,)),
        pltpu.SemaphoreType.DMA((2,)),
        pltpu.VMEM_SHARED((NPAD, D), jnp.float32),
    ],
)


def _dense_body(x_ref, w_ref, gamma_ref, beta_ref, deg_ref, p_ref):
  x = x_ref[...]
  mean = jnp.mean(x, axis=0)
  var = jnp.mean((x - mean) ** 2, axis=0)
  h = (x - mean) * lax.rsqrt(var + 1e-5) * gamma_ref[...] + beta_ref[...]
  h = jnp.maximum(h, 0.0)
  deg_src = deg_ref[0, 0, :] + deg_ref[1, 0, :]
  norm_src = jnp.where(deg_src > 0.0, lax.rsqrt(jnp.maximum(deg_src, 1.0)), 0.0)
  h = h * norm_src[:N, None]
  p = jnp.dot(h, w_ref[...], preferred_element_type=jnp.float32)
  p_ref[...] = jnp.concatenate(
      [p, jnp.zeros((NPAD - N, D), jnp.float32)], axis=0)


def _dense_kernel(x, W, gamma, beta, deg):
  return pl.pallas_call(
      _dense_body,
      out_shape=jax.ShapeDtypeStruct((NPAD, D), jnp.float32),
  )(x, W, gamma, beta, deg)


def _final_body(x_ref, acc_ref, deg_ref, b_ref, out_ref):
  deg_dst = deg_ref[0, 1, :] + deg_ref[1, 1, :]
  norm_dst = jnp.where(deg_dst > 0.0, lax.rsqrt(jnp.maximum(deg_dst, 1.0)), 0.0)
  agg = acc_ref[0, :N] + acc_ref[1, :N]
  out_ref[...] = x_ref[...] + agg * norm_dst[:N, None] + b_ref[...]


def _final_kernel(x, acc, deg, b):
  return pl.pallas_call(
      _final_body,
      out_shape=jax.ShapeDtypeStruct((N, D), jnp.float32),
  )(x, acc, deg, b)


@jax.jit
def kernel(node_feats, edge_index, W, b, gamma, beta):
  ei = edge_index.astype(jnp.int32)
  src2 = ei[0].reshape(EROWS, CH)
  dst2 = ei[1].reshape(EROWS, CH)
  pad = jnp.broadcast_to(N + jnp.arange(NPAD - N, dtype=jnp.int32),
                         (NW, NPAD - N))
  src3 = jnp.concatenate([ei[0].reshape(NW, E // NW), pad],
                         axis=1).reshape(NW, NCHUNK, CH)
  dst3 = jnp.concatenate([ei[1].reshape(NW, E // NW), pad],
                         axis=1).reshape(NW, NCHUNK, CH)
  deg = _deg_kernel(src2, dst2)
  p = _dense_kernel(node_feats, W, gamma, beta, deg)
  acc = _scatter_kernel(p, src3, dst3)
  return _final_kernel(node_feats, acc, deg, b)


# speedup vs baseline: 1.0155x; 1.0055x over previous
import jax
import jax.numpy as jnp
from jax import lax
from jax.experimental import pallas as pl
from jax.experimental.pallas import tpu as pltpu
from jax.experimental.pallas import tpu_sc as plsc

N = 10000
E = 320000
D = 128

NC = 2
NS = 16
NW = NC * NS

NPAD = 10240
CH = 128
NCHUNK = 80
E_PER_W = NCHUNK * CH
EPAD = NW * E_PER_W

NSLAB = 10
SLAB = NCHUNK // NSLAB

DEG_PER_TILE = NPAD // NS
ROWS_PER_TILE = NPAD // NS
ZROWS = 128


def _zero_1d(ref, nwords):
  zv = jnp.zeros((16,), jnp.float32)

  def body(i, _):
    ref[pl.ds(i * 16, 16)] = zv
    return 0

  lax.fori_loop(0, nwords // 16, body, 0)


def _zero_2d(ref, nrows):
  zv = jnp.zeros((16,), jnp.float32)

  def body(i, _):
    def inner(j, _):
      ref[i, pl.ds(j * 16, 16)] = zv
      return 0

    lax.fori_loop(0, 8, inner, 0)
    return 0

  lax.fori_loop(0, nrows, body, 0)


EROWS = E // CH
FULL_G = 312


def _deg_body(src_ref, dst_ref, out_ref, sidx, didx, sx, dx, ones_v, zb,
              sems, dsrc_sh, ddst_sh):
  cid = lax.axis_index("c")
  sid = lax.axis_index("s")
  wid = sid * NC + cid

  ov = jnp.ones((16,), jnp.float32)
  for k in range(CH // 16):
    ones_v[pl.ds(k * 16, 16)] = ov

  lt24 = wid < 24
  n_g = jnp.where(lt24, 10, 9)
  g0 = jnp.where(lt24, 10 * wid, 240 + 9 * (wid - 24))
  row0 = pl.multiple_of(g0 * 8, 8)

  pltpu.sync_copy(src_ref.at[pl.ds(row0, 72)], sidx.at[pl.ds(0, 72)])
  pltpu.sync_copy(dst_ref.at[pl.ds(row0, 72)], didx.at[pl.ds(0, 72)])

  @pl.when(lt24)
  def _load_rest():
    r = pl.multiple_of(row0 + 72, 8)
    pltpu.sync_copy(src_ref.at[pl.ds(r, 8)], sidx.at[pl.ds(72, 8)])
    pltpu.sync_copy(dst_ref.at[pl.ds(r, 8)], didx.at[pl.ds(72, 8)])

  @pl.when(wid < 4)
  def _load_extra():
    pltpu.sync_copy(src_ref.at[pl.ds(FULL_G * 8, 4)], sx)
    pltpu.sync_copy(dst_ref.at[pl.ds(FULL_G * 8, 4)], dx)

  _zero_1d(zb, DEG_PER_TILE)
  off = pl.multiple_of(sid * DEG_PER_TILE, 8)
  pltpu.sync_copy(zb, dsrc_sh.at[pl.ds(off, DEG_PER_TILE)])
  pltpu.sync_copy(zb, ddst_sh.at[pl.ds(off, DEG_PER_TILE)])
  plsc.subcore_barrier()

  def group(g, _):
    hs = []
    for j in range(SLAB):
      c = g * SLAB + j
      hs.append(pltpu.async_copy(ones_v, dsrc_sh.at[sidx.at[c]],
                                 sems.at[j], add=True))
      hs.append(pltpu.async_copy(ones_v, ddst_sh.at[didx.at[c]],
                                 sems.at[SLAB + j], add=True))
    for h in hs:
      h.wait()
    return 0

  lax.fori_loop(0, n_g, group, 0)

  @pl.when(wid < 4)
  def _extra_chunk():
    pltpu.sync_copy(ones_v, dsrc_sh.at[sx.at[wid]], add=True)
    pltpu.sync_copy(ones_v, ddst_sh.at[dx.at[wid]], add=True)

  plsc.subcore_barrier()

  pltpu.sync_copy(dsrc_sh.at[pl.ds(off, DEG_PER_TILE)],
                  out_ref.at[cid, 0, pl.ds(off, DEG_PER_TILE)])
  pltpu.sync_copy(ddst_sh.at[pl.ds(off, DEG_PER_TILE)],
                  out_ref.at[cid, 1, pl.ds(off, DEG_PER_TILE)])


_deg_kernel = pl.kernel(
    _deg_body,
    out_type=jax.ShapeDtypeStruct((NC, 2, NPAD), jnp.float32),
    mesh=plsc.VectorSubcoreMesh(core_axis_name="c", subcore_axis_name="s"),
    scratch_types=[
        pltpu.VMEM((NCHUNK, CH), jnp.int32),
        pltpu.VMEM((NCHUNK, CH), jnp.int32),
        pltpu.VMEM((4, CH), jnp.int32),
        pltpu.VMEM((4, CH), jnp.int32),
        pltpu.VMEM((CH,), jnp.float32),
        pltpu.VMEM((DEG_PER_TILE,), jnp.float32),
        pltpu.SemaphoreType.DMA((2 * SLAB,)),
        pltpu.VMEM_SHARED((NPAD,), jnp.float32),
        pltpu.VMEM_SHARED((NPAD,), jnp.float32),
    ],
)


def _scatter_body(p_ref, src_ref, dst_ref, out_ref, sidx, didx, rows_v,
                  isem, gsem, ssem, acc_sh):
  cid = lax.axis_index("c")
  sid = lax.axis_index("s")
  wid = sid * NC + cid

  pltpu.sync_copy(dst_ref.at[wid], didx)
  pltpu.sync_copy(src_ref.at[wid, pl.ds(0, SLAB)], sidx.at[0])

  _zero_2d(rows_v.at[0], ZROWS)
  for k in range(ROWS_PER_TILE // ZROWS):
    r0 = pl.multiple_of(sid * ROWS_PER_TILE + k * ZROWS, 8)
    pltpu.sync_copy(rows_v.at[0], acc_sh.at[pl.ds(r0, ZROWS)])
  plsc.subcore_barrier()

  def _wait_scatter(c):
    pltpu.make_async_copy(rows_v.at[c % 2],
                          acc_sh.at[didx.at[c]],
                          ssem.at[c % 2]).wait()

  def _wait_gather(s, j):
    c = s * SLAB + j
    pltpu.make_async_copy(p_ref.at[sidx.at[s % 2, j]],
                          rows_v.at[c % 2],
                          gsem.at[c % 2]).wait()

  def slab_body(s, _):
    ps = s % 2

    @pl.when(s > 0)
    def _wait_idx():
      pltpu.make_async_copy(src_ref.at[wid, pl.ds(s * SLAB, SLAB)],
                            sidx.at[ps], isem.at[ps]).wait()

    for j in range(SLAB):
      b = j % 2
      c = s * SLAB + j

      @pl.when(c >= 2)
      def _ws(c=c):
        _wait_scatter(c - 2)

      pltpu.async_copy(p_ref.at[sidx.at[ps, j]], rows_v.at[b], gsem.at[b])

      @pl.when(c >= 1)
      def _wg(s=s, j=j, c=c):
        if j == 0:
          _wait_gather(s - 1, SLAB - 1)
        else:
          _wait_gather(s, j - 1)
        pltpu.async_copy(rows_v.at[1 - b], acc_sh.at[didx.at[c - 1]],
                         ssem.at[(c - 1) % 2], add=True)

      if j == 0:
        @pl.when(s < NSLAB - 1)
        def _prefetch_idx():
          pltpu.async_copy(src_ref.at[wid, pl.ds((s + 1) * SLAB, SLAB)],
                           sidx.at[1 - ps], isem.at[1 - ps])

    return 0

  lax.fori_loop(0, NSLAB, slab_body, 0)

  last = NCHUNK - 1
  _wait_gather(NSLAB - 1, SLAB - 1)
  pltpu.sync_copy(rows_v.at[last % 2], acc_sh.at[didx.at[last]], add=True)
  _wait_scatter(last - 1)
  plsc.subcore_barrier()

  for k in range(ROWS_PER_TILE // ZROWS):
    r0 = pl.multiple_of(sid * ROWS_PER_TILE + k * ZROWS, 8)
    pltpu.sync_copy(acc_sh.at[pl.ds(r0, ZROWS)],
                    out_ref.at[cid, pl.ds(r0, ZROWS)])


_scatter_kernel = pl.kernel(
    _scatter_body,
    out_type=jax.ShapeDtypeStruct((NC, NPAD, D), jnp.float32),
    mesh=plsc.VectorSubcoreMesh(core_axis_name="c", subcore_axis_name="s"),
    scratch_types=[
        pltpu.VMEM((2, SLAB, CH), jnp.int32),
        pltpu.VMEM((NCHUNK, CH), jnp.int32),
        pltpu.VMEM((2, ZROWS, D), jnp.float32),
        pltpu.SemaphoreType.DMA((2,)),
        pltpu.SemaphoreType.DMA((2,)),
        pltpu.SemaphoreType.DMA((2,)),
        pltpu.VMEM_SHARED((NPAD, D), jnp.float32),
    ],
)


def _bn_body(x_ref, gamma_ref, beta_ref, h_ref):
  x = x_ref[...]
  mean = jnp.mean(x, axis=0)
  var = jnp.mean((x - mean) ** 2, axis=0)
  h = (x - mean) * lax.rsqrt(var + 1e-5) * gamma_ref[...] + beta_ref[...]
  h_ref[...] = jnp.maximum(h, 0.0)


def _bn_kernel(x, gamma, beta):
  return pl.pallas_call(
      _bn_body,
      out_shape=jax.ShapeDtypeStruct((N, D), jnp.float32),
  )(x, gamma, beta)


def _dense_body(h_ref, w_ref, deg_ref, p_ref):
  deg_src = deg_ref[0, 0, :] + deg_ref[1, 0, :]
  norm_src = jnp.where(deg_src > 0.0, lax.rsqrt(jnp.maximum(deg_src, 1.0)), 0.0)
  h = h_ref[...] * norm_src[:N, None]
  p = jnp.dot(h, w_ref[...], preferred_element_type=jnp.float32)
  p_ref[...] = jnp.concatenate(
      [p, jnp.zeros((NPAD - N, D), jnp.float32)], axis=0)


def _dense_kernel(h, W, deg):
  return pl.pallas_call(
      _dense_body,
      out_shape=jax.ShapeDtypeStruct((NPAD, D), jnp.float32),
  )(h, W, deg)


def _final_body(x_ref, acc_ref, deg_ref, b_ref, out_ref):
  deg_dst = deg_ref[0, 1, :] + deg_ref[1, 1, :]
  norm_dst = jnp.where(deg_dst > 0.0, lax.rsqrt(jnp.maximum(deg_dst, 1.0)), 0.0)
  agg = acc_ref[0, :N] + acc_ref[1, :N]
  out_ref[...] = x_ref[...] + agg * norm_dst[:N, None] + b_ref[...]


def _final_kernel(x, acc, deg, b):
  return pl.pallas_call(
      _final_body,
      out_shape=jax.ShapeDtypeStruct((N, D), jnp.float32),
  )(x, acc, deg, b)


@jax.jit
def kernel(node_feats, edge_index, W, b, gamma, beta):
  ei = edge_index.astype(jnp.int32)
  src2 = ei[0].reshape(EROWS, CH)
  dst2 = ei[1].reshape(EROWS, CH)
  pad = jnp.broadcast_to(N + jnp.arange(NPAD - N, dtype=jnp.int32),
                         (NW, NPAD - N))
  src3 = jnp.concatenate([ei[0].reshape(NW, E // NW), pad],
                         axis=1).reshape(NW, NCHUNK, CH)
  dst3 = jnp.concatenate([ei[1].reshape(NW, E // NW), pad],
                         axis=1).reshape(NW, NCHUNK, CH)
  deg = _deg_kernel(src2, dst2)
  h = _bn_kernel(node_feats, gamma, beta)
  p = _dense_kernel(h, W, deg)
  acc = _scatter_kernel(p, src3, dst3)
  return _final_kernel(node_feats, acc, deg, b)
